# baseline (speedup 1.0000x reference)
import functools

import jax
import jax.numpy as jnp
from jax import lax
from jax.experimental import pallas as pl
from jax.experimental.pallas import tpu as pltpu
from jax.experimental.pallas import tpu_sc as plsc

B = 1024
T = 200
C = 256
H = 2
VPAD = 128
NW = 32
B_PER = B // NW
CCHUNK = 128
ROW = H * T
W0S = tuple(range(0, T - 16, 16)) + (T - 16,)


def _table_body(emb_ref, w_ref, b_ref, out_ref):
    out_ref[...] = lax.dot_general(
        w_ref[...], emb_ref[...],
        dimension_numbers=(((0,), (1,)), ((), ())),
        preferred_element_type=jnp.float32,
    ) + b_ref[...]


def _fused_table(emb_pad, w_lin, b_col):
    return pl.pallas_call(
        _table_body,
        out_shape=jax.ShapeDtypeStruct((C, VPAD), jnp.float32),
    )(emb_pad, w_lin, b_col)


@functools.partial(
    pl.kernel,
    mesh=plsc.VectorSubcoreMesh(core_axis_name="c", subcore_axis_name="s"),
    out_type=jax.ShapeDtypeStruct((B * C * H * T,), jnp.float32),
    compiler_params=pltpu.CompilerParams(needs_layout_passes=False),
    scratch_types=[
        pltpu.VMEM((C * VPAD,), jnp.float32),
        pltpu.VMEM((T,), jnp.int32),
        pltpu.VMEM((CCHUNK * ROW,), jnp.float32),
    ],
)
def _sc_lookup(tt_hbm, x_hbm, out_hbm, tt_v, x_v, stg_v):
    wid = lax.axis_index("s") * 2 + lax.axis_index("c")
    pltpu.sync_copy(tt_hbm, tt_v)

    def b_body(i, carry):
        b = wid * B_PER + i
        pltpu.sync_copy(x_hbm.at[pl.ds(b * T, T)], x_v)
        xvecs = [x_v[pl.ds(w0, 16)] for w0 in W0S]
        for c0 in range(0, C, CCHUNK):
            def c_body(c, inner):
                base = jnp.full((16,), (c0 + c) * VPAD, jnp.int32)
                off = c * ROW
                for k, w0 in enumerate(W0S):
                    row = plsc.load_gather(tt_v, [xvecs[k] + base])
                    stg_v[pl.ds(off + w0, 16)] = row
                    stg_v[pl.ds(off + T + w0, 16)] = row
                return inner
            lax.fori_loop(0, CCHUNK, c_body, 0)
            dst = out_hbm.at[pl.ds(b * (C * ROW) + c0 * ROW, CCHUNK * ROW)]
            pltpu.sync_copy(stg_v, dst)
        return carry

    lax.fori_loop(0, B_PER, b_body, 0)


def kernel(x, f_xs_shape, embed_table, W_lin, b_lin):
    emb_pad = jnp.zeros((VPAD, embed_table.shape[1]), jnp.float32)
    emb_pad = emb_pad.at[: embed_table.shape[0]].set(embed_table)
    tt = _fused_table(emb_pad, W_lin, b_lin.reshape(C, 1))
    out = _sc_lookup(tt.reshape(-1), x.reshape(-1))
    return out.reshape(B, C, H, T)


# speedup vs baseline: 3.0287x; 3.0287x over previous
import functools

import jax
import jax.numpy as jnp
from jax import lax
from jax.experimental import pallas as pl
from jax.experimental.pallas import tpu as pltpu
from jax.experimental.pallas import tpu_sc as plsc

B = 1024
T = 200
C = 256
H = 2
VPAD = 128
NW = 32
B_PER = B // NW
CCHUNK = 128
ROW = H * T
W0S = tuple(range(0, T - 16, 16)) + (T - 16,)


def _table_body(emb_ref, w_ref, b_ref, out_ref):
    out_ref[...] = lax.dot_general(
        w_ref[...], emb_ref[...],
        dimension_numbers=(((0,), (1,)), ((), ())),
        preferred_element_type=jnp.float32,
    ) + b_ref[...]


def _fused_table(emb_pad, w_lin, b_col):
    return pl.pallas_call(
        _table_body,
        out_shape=jax.ShapeDtypeStruct((C, VPAD), jnp.float32),
    )(emb_pad, w_lin, b_col)


NCHUNK = C // CCHUNK


@functools.partial(
    pl.kernel,
    mesh=plsc.VectorSubcoreMesh(core_axis_name="c", subcore_axis_name="s"),
    out_type=jax.ShapeDtypeStruct((B, C, H, T), jnp.float32),
    compiler_params=pltpu.CompilerParams(needs_layout_passes=False),
    scratch_types=[
        pltpu.VMEM((C * VPAD,), jnp.float32),
        pltpu.VMEM((B_PER * T,), jnp.int32),
        pltpu.VMEM((NCHUNK * CCHUNK, 1, T), jnp.float32),
        pltpu.SemaphoreType.DMA((NCHUNK,)),
    ],
)
def _sc_lookup(tt_hbm, x_hbm, out_hbm, tt_v, x_v, stg_v, sems):
    wid = lax.axis_index("s") * 2 + lax.axis_index("c")
    pltpu.sync_copy(tt_hbm, tt_v)
    pltpu.sync_copy(x_hbm.at[pl.ds(wid * (B_PER * T), B_PER * T)], x_v)

    def b_body(i, carry):
        b = wid * B_PER + i
        xvecs = [x_v[pl.ds(i * T + w0, 16)] for w0 in W0S]
        for k in range(NCHUNK):
            c0 = k * CCHUNK
            src = stg_v.at[pl.ds(k * CCHUNK, CCHUNK)]
            dsts = [out_hbm.at[b, pl.ds(c0, CCHUNK), pl.ds(h, 1)]
                    for h in range(H)]

            @pl.when(i > 0)
            def _drain():
                for h in range(H):
                    pltpu.make_async_copy(src, dsts[h], sems.at[k]).wait()

            @plsc.parallel_loop(0, CCHUNK, 1, unroll=8)
            def _fill(c):
                base = jnp.full((16,), (c0 + c) * VPAD, jnp.int32)
                row = k * CCHUNK + c
                for j, w0 in enumerate(W0S):
                    stg_v[row, 0, pl.ds(w0, 16)] = plsc.load_gather(
                        tt_v, [xvecs[j] + base])

            for h in range(H):
                pltpu.async_copy(src, dsts[h], sems.at[k])
        return carry

    lax.fori_loop(0, B_PER, b_body, 0)

    last_b = wid * B_PER + B_PER - 1
    for k in range(NCHUNK):
        src = stg_v.at[pl.ds(k * CCHUNK, CCHUNK)]
        for h in range(H):
            pltpu.make_async_copy(
                src, out_hbm.at[last_b, pl.ds(k * CCHUNK, CCHUNK), pl.ds(h, 1)],
                sems.at[k]).wait()


def kernel(x, f_xs_shape, embed_table, W_lin, b_lin):
    emb_pad = jnp.zeros((VPAD, embed_table.shape[1]), jnp.float32)
    emb_pad = emb_pad.at[: embed_table.shape[0]].set(embed_table)
    tt = _fused_table(emb_pad, W_lin, b_lin.reshape(C, 1))
    return _sc_lookup(tt.reshape(-1), x.reshape(-1))


# speedup vs baseline: 5.2057x; 1.7188x over previous
import functools

import jax
import jax.numpy as jnp
from jax import lax
from jax.experimental import pallas as pl
from jax.experimental.pallas import tpu as pltpu
from jax.experimental.pallas import tpu_sc as plsc

B = 1024
T = 200
C = 256
H = 2
VPAD = 128
NW = 32
B_PER = B // NW
TSPLIT = (0, 128)
TLEN = (128, T - 128)


def _table_body(emb_ref, w_ref, b_ref, out_ref):
    out_ref[...] = lax.dot_general(
        emb_ref[...], w_ref[...],
        dimension_numbers=(((1,), (0,)), ((), ())),
        preferred_element_type=jnp.float32,
    ) + b_ref[...]


def _fused_table(emb_pad, w_lin, b_row):
    return pl.pallas_call(
        _table_body,
        out_shape=jax.ShapeDtypeStruct((VPAD, C), jnp.float32),
    )(emb_pad, w_lin, b_row)


@functools.partial(
    pl.kernel,
    mesh=plsc.VectorSubcoreMesh(core_axis_name="c", subcore_axis_name="s"),
    out_type=jax.ShapeDtypeStruct((B, H, T, C), jnp.float32),
    compiler_params=pltpu.CompilerParams(needs_layout_passes=False),
    scratch_types=[
        pltpu.VMEM((B_PER * T,), jnp.int32),
        pltpu.VMEM((2, T, C), jnp.float32),
        pltpu.SemaphoreType.DMA((2,)),
        pltpu.SemaphoreType.DMA((2,)),
    ],
)
def _sc_lookup(tv_hbm, x_hbm, out_hbm, x_v, stg_v, gsem, wsem):
    wid = lax.axis_index("s") * 2 + lax.axis_index("c")
    pltpu.sync_copy(x_hbm.at[pl.ds(wid * (B_PER * T), B_PER * T)], x_v)

    def pair_body(i2, carry):
        for p in range(2):
            i = i2 * 2 + p
            b = wid * B_PER + i
            stg = stg_v.at[p]
            writes = [
                pltpu.make_async_copy(stg, out_hbm.at[b, h], wsem.at[p])
                for h in range(H)
            ]

            @pl.when(i2 > 0)
            def _drain():
                for w in writes:
                    w.wait()

            gathers = [
                pltpu.make_async_copy(
                    tv_hbm.at[x_v.at[pl.ds(i * T + off, ln)]],
                    stg_v.at[p, pl.ds(off, ln)],
                    gsem.at[p],
                )
                for off, ln in zip(TSPLIT, TLEN)
            ]
            for g in gathers:
                g.start()
            for g in gathers:
                g.wait()

            for w in writes:
                w.start()
        return carry

    lax.fori_loop(0, B_PER // 2, pair_body, 0)

    last = wid * B_PER + B_PER - 2
    for p in range(2):
        for h in range(H):
            pltpu.make_async_copy(
                stg_v.at[p], out_hbm.at[last + p, h], wsem.at[p]).wait()


def kernel(x, f_xs_shape, embed_table, W_lin, b_lin):
    emb_pad = jnp.zeros((VPAD, embed_table.shape[1]), jnp.float32)
    emb_pad = emb_pad.at[: embed_table.shape[0]].set(embed_table)
    tv = _fused_table(emb_pad, W_lin, b_lin.reshape(1, C))
    out = _sc_lookup(tv, x.reshape(-1))
    return jnp.transpose(out, (0, 3, 1, 2))


# speedup vs baseline: 18.1581x; 3.4881x over previous
import functools

import jax
import jax.numpy as jnp
from jax import lax
from jax.experimental import pallas as pl
from jax.experimental.pallas import tpu as pltpu
from jax.experimental.pallas import tpu_sc as plsc

B = 1024
T = 200
C = 256
H = 2
VPAD = 104
NW = 32
B_PER = B // NW


def _table_body(emb_ref, w_ref, b_ref, out_ref):
    out_ref[...] = lax.dot_general(
        emb_ref[...], w_ref[...],
        dimension_numbers=(((1,), (0,)), ((), ())),
        preferred_element_type=jnp.float32,
    ) + b_ref[...]


def _fused_table(emb_pad, w_lin, b_row):
    return pl.pallas_call(
        _table_body,
        out_shape=jax.ShapeDtypeStruct((VPAD, C), jnp.float32),
    )(emb_pad, w_lin, b_row)


@functools.partial(
    pl.kernel,
    mesh=plsc.VectorSubcoreMesh(core_axis_name="c", subcore_axis_name="s"),
    out_type=jax.ShapeDtypeStruct((B, H, T, C), jnp.float32),
    compiler_params=pltpu.CompilerParams(needs_layout_passes=False),
    scratch_types=[
        pltpu.VMEM((T,), jnp.int32),
        pltpu.VMEM((T,), jnp.int32),
        pltpu.VMEM((2, T, C), jnp.float32),
        pltpu.VMEM((VPAD * C,), jnp.float32),
        pltpu.SemaphoreType.DMA((2,)),
        pltpu.SemaphoreType.DMA((2,)),
    ],
)
def _sc_lookup(tv_hbm, x_hbm, out_hbm, xa_v, xb_v, stg_v, tv_v, xsem, wsem):
    wid = lax.axis_index("s") * 2 + lax.axis_index("c")
    b0 = wid * B_PER
    civ = lax.iota(jnp.int32, 16)
    pltpu.sync_copy(tv_hbm, tv_v)
    xbufs = (xa_v, xb_v)
    pltpu.async_copy(x_hbm.at[pl.ds(b0 * T, T)], xa_v, xsem.at[0])

    def xload(i, p):
        nxt = jnp.minimum(i + 1, B_PER - 1)
        pltpu.async_copy(
            x_hbm.at[pl.ds((b0 + nxt) * T, T)], xbufs[1 - p], xsem.at[1 - p])

    def pair_body(i2, carry):
        for p in range(2):
            i = i2 * 2 + p
            b = b0 + i
            stg = stg_v.at[p]
            writes = [
                pltpu.make_async_copy(stg, out_hbm.at[b, h], wsem.at[p])
                for h in range(H)
            ]

            pltpu.make_async_copy(
                x_hbm.at[pl.ds(b * T, T)], xbufs[p], xsem.at[p]).wait()
            xload(i, p)

            @pl.when(i2 > 0)
            def _drain():
                for w in writes:
                    w.wait()

            @plsc.parallel_loop(0, T, 1, unroll=2)
            def _fill(w):
                xsp = plsc.load_gather(xbufs[p], [jnp.full((16,), w, jnp.int32)])
                base = xsp * C
                for k in range(C // 16):
                    lanes = base + (civ + (k * 16))
                    stg_v[p, w, pl.ds(k * 16, 16)] = plsc.load_gather(
                        tv_v, [lanes])

            for w in writes:
                w.start()
        return carry

    lax.fori_loop(0, B_PER // 2, pair_body, 0)

    pltpu.make_async_copy(
        x_hbm.at[pl.ds((b0 + B_PER - 1) * T, T)], xa_v, xsem.at[0]).wait()
    last = b0 + B_PER - 2
    for p in range(2):
        for h in range(H):
            pltpu.make_async_copy(
                stg_v.at[p], out_hbm.at[last + p, h], wsem.at[p]).wait()


def kernel(x, f_xs_shape, embed_table, W_lin, b_lin):
    emb_pad = jnp.zeros((VPAD, embed_table.shape[1]), jnp.float32)
    emb_pad = emb_pad.at[: embed_table.shape[0]].set(embed_table)
    tv = _fused_table(emb_pad, W_lin, b_lin.reshape(1, C))
    out = _sc_lookup(tv.reshape(-1), x.reshape(-1))
    return jnp.transpose(out, (0, 3, 1, 2))
